# baseline (device time: 261834 ns/iter reference)
import jax
import jax.numpy as jnp
from jax import lax
from jax.experimental import pallas as pl
from jax.experimental.pallas import tpu as pltpu

K = 2048
M_SHARD = 1024
M_BLK = 256
F = 8192
DY_CHUNK = 512
N_DY = F // DY_CHUNK
CW = 512
NC = F // CW
SUB = CW // DY_CHUNK


def kernel(x, dy):
    my_x = lax.axis_index("x")
    my_y = lax.axis_index("y")
    my_z = lax.axis_index("z")
    q = 2 * my_z + my_x

    col_me = my_y * M_SHARD + q * M_BLK
    col_pr = (1 - my_y) * M_SHARD + q * M_BLK
    a_me = lax.dynamic_slice(x, (0, col_me), (K, M_BLK)).T
    a_pr = lax.dynamic_slice(x, (0, col_pr), (K, M_BLK)).T

    def body(a_me_ref, a_pr_ref, dy_ref, out_ref,
             dy_vmem, send_buf, recv_buf,
             dy_sems, y_send_sems, y_recv_sems, x_send_sems, x_recv_sems,
             z_send_sems, z_recv_sems):
        my_x = lax.axis_index("x")
        my_y = lax.axis_index("y")
        my_z = lax.axis_index("z")
        q = 2 * my_z + my_x
        row0 = q * M_BLK
        row0x = (2 * my_z + (1 - my_x)) * M_BLK

        y_dev = (my_x, 1 - my_y, my_z)
        x_dev = (1 - my_x, my_y, my_z)
        z_dev = (my_x, my_y, 1 - my_z)

        barrier = pltpu.get_barrier_semaphore()
        for dev in (y_dev, x_dev, z_dev):
            pl.semaphore_signal(barrier, inc=1, device_id=dev,
                                device_id_type=pl.DeviceIdType.MESH)
        pl.semaphore_wait(barrier, 3)

        def make_y(c):
            cols = pl.ds(c * CW, CW)
            return pltpu.make_async_remote_copy(
                src_ref=send_buf.at[:, cols], dst_ref=recv_buf.at[:, cols],
                send_sem=y_send_sems.at[c], recv_sem=y_recv_sems.at[c],
                device_id=y_dev, device_id_type=pl.DeviceIdType.MESH)

        def make_x(c):
            cols = pl.ds(c * CW, CW)
            return pltpu.make_async_remote_copy(
                src_ref=out_ref.at[pl.ds(row0, M_BLK), cols],
                dst_ref=out_ref.at[pl.ds(row0, M_BLK), cols],
                send_sem=x_send_sems.at[c], recv_sem=x_recv_sems.at[c],
                device_id=x_dev, device_id_type=pl.DeviceIdType.MESH)

        def make_z(c, rows, sem_idx):
            cols = pl.ds(c * CW, CW)
            return pltpu.make_async_remote_copy(
                src_ref=out_ref.at[pl.ds(rows, M_BLK), cols],
                dst_ref=out_ref.at[pl.ds(rows, M_BLK), cols],
                send_sem=z_send_sems.at[sem_idx], recv_sem=z_recv_sems.at[sem_idx],
                device_id=z_dev, device_id_type=pl.DeviceIdType.MESH)

        y_rdmas = [make_y(c) for c in range(NC)]
        x_rdmas = [make_x(c) for c in range(NC)]
        z1_rdmas = [make_z(c, row0, c) for c in range(NC)]
        z2_rdmas = [make_z(c, row0x, NC + c) for c in range(NC)]

        def dy_copy(i):
            cols = pl.ds(i * DY_CHUNK, DY_CHUNK)
            return pltpu.make_async_copy(
                dy_ref.at[:, cols], dy_vmem.at[i % 2], dy_sems.at[i % 2])

        copies = [dy_copy(i) for i in range(N_DY)]
        copies[0].start()
        for i in range(N_DY):
            if i + 1 < N_DY:
                copies[i + 1].start()
            copies[i].wait()
            cols = pl.ds(i * DY_CHUNK, DY_CHUNK)
            d = dy_vmem[i % 2, :, :]
            send_buf[:, cols] = lax.dot_general(
                a_pr_ref[:, :], d, (((1,), (0,)), ((), ())),
                preferred_element_type=jnp.float32)
            out_ref[pl.ds(row0, M_BLK), cols] = lax.dot_general(
                a_me_ref[:, :], d, (((1,), (0,)), ((), ())),
                preferred_element_type=jnp.float32)
            if (i + 1) % SUB == 0:
                y_rdmas[(i + 1) // SUB - 1].start()

        LAG = 2
        for c in range(NC):
            cols = pl.ds(c * CW, CW)
            y_rdmas[c].wait_recv()
            out_ref[pl.ds(row0, M_BLK), cols] = (
                out_ref[pl.ds(row0, M_BLK), cols] + recv_buf[:, cols])
            z1_rdmas[c].start()
            x_rdmas[c].start()
            if c >= LAG:
                x_rdmas[c - LAG].wait_recv()
                z2_rdmas[c - LAG].start()

        for c in range(NC - LAG, NC):
            x_rdmas[c].wait_recv()
            z2_rdmas[c].start()

        for c in range(NC):
            z1_rdmas[c].wait_recv()
            z2_rdmas[c].wait_recv()
        for c in range(NC):
            y_rdmas[c].wait_send()
            x_rdmas[c].wait_send()
            z1_rdmas[c].wait_send()
            z2_rdmas[c].wait_send()

    return pl.pallas_call(
        body,
        out_shape=jax.ShapeDtypeStruct((M_SHARD, F), jnp.float32),
        in_specs=[
            pl.BlockSpec(memory_space=pltpu.VMEM),
            pl.BlockSpec(memory_space=pltpu.VMEM),
            pl.BlockSpec(memory_space=pl.ANY),
        ],
        out_specs=pl.BlockSpec(memory_space=pltpu.VMEM),
        scratch_shapes=[
            pltpu.VMEM((2, K, DY_CHUNK), jnp.float32),
            pltpu.VMEM((M_BLK, F), jnp.float32),
            pltpu.VMEM((M_BLK, F), jnp.float32),
            pltpu.SemaphoreType.DMA((2,)),
            pltpu.SemaphoreType.DMA((NC,)),
            pltpu.SemaphoreType.DMA((NC,)),
            pltpu.SemaphoreType.DMA((NC,)),
            pltpu.SemaphoreType.DMA((NC,)),
            pltpu.SemaphoreType.DMA((2 * NC,)),
            pltpu.SemaphoreType.DMA((2 * NC,)),
        ],
        compiler_params=pltpu.CompilerParams(
            collective_id=0, vmem_limit_bytes=64 * 1024 * 1024),
    )(a_me, a_pr, dy)


# device time: 144619 ns/iter; 1.8105x vs baseline; 1.8105x over previous
import os

import jax
import jax.numpy as jnp
from jax import lax
from jax.experimental import pallas as pl
from jax.experimental.pallas import tpu as pltpu

K = 2048
M_SHARD = 1024
M_BLK = 256
F = 8192
DY_CHUNK = 512
N_DY = F // DY_CHUNK
CW = 512
NC = F // CW
SUB = CW // DY_CHUNK

DO_Y = "y" not in os.environ.get("KSKIP", "")
DO_X = "x" not in os.environ.get("KSKIP", "")
DO_Z = "z" not in os.environ.get("KSKIP", "")


def kernel(x, dy):
    my_x = lax.axis_index("x")
    my_y = lax.axis_index("y")
    my_z = lax.axis_index("z")
    q = 2 * my_z + my_x

    col_me = my_y * M_SHARD + q * M_BLK
    col_pr = (1 - my_y) * M_SHARD + q * M_BLK
    a_me = lax.dynamic_slice(x, (0, col_me), (K, M_BLK)).T
    a_pr = lax.dynamic_slice(x, (0, col_pr), (K, M_BLK)).T

    def body(a_me_ref, a_pr_ref, dy_ref, out_ref,
             dy_vmem, send_buf, recv_buf,
             dy_sems, y_send_sems, y_recv_sems, x_send_sems, x_recv_sems,
             z_send_sems, z_recv_sems):
        my_x = lax.axis_index("x")
        my_y = lax.axis_index("y")
        my_z = lax.axis_index("z")
        q = 2 * my_z + my_x
        row0 = q * M_BLK
        row0x = (2 * my_z + (1 - my_x)) * M_BLK

        y_dev = (my_x, 1 - my_y, my_z)
        x_dev = (1 - my_x, my_y, my_z)
        z_dev = (my_x, my_y, 1 - my_z)

        barrier = pltpu.get_barrier_semaphore()
        for dev in (y_dev, x_dev, z_dev):
            pl.semaphore_signal(barrier, inc=1, device_id=dev,
                                device_id_type=pl.DeviceIdType.MESH)
        pl.semaphore_wait(barrier, 3)

        def make_y(c):
            cols = pl.ds(c * CW, CW)
            return pltpu.make_async_remote_copy(
                src_ref=send_buf.at[:, cols], dst_ref=recv_buf.at[:, cols],
                send_sem=y_send_sems.at[c], recv_sem=y_recv_sems.at[c],
                device_id=y_dev, device_id_type=pl.DeviceIdType.MESH)

        def make_x(c):
            cols = pl.ds(c * CW, CW)
            return pltpu.make_async_remote_copy(
                src_ref=out_ref.at[pl.ds(row0, M_BLK), cols],
                dst_ref=out_ref.at[pl.ds(row0, M_BLK), cols],
                send_sem=x_send_sems.at[c], recv_sem=x_recv_sems.at[c],
                device_id=x_dev, device_id_type=pl.DeviceIdType.MESH)

        def make_z(c, rows, sem_idx):
            cols = pl.ds(c * CW, CW)
            return pltpu.make_async_remote_copy(
                src_ref=out_ref.at[pl.ds(rows, M_BLK), cols],
                dst_ref=out_ref.at[pl.ds(rows, M_BLK), cols],
                send_sem=z_send_sems.at[sem_idx], recv_sem=z_recv_sems.at[sem_idx],
                device_id=z_dev, device_id_type=pl.DeviceIdType.MESH)

        y_rdmas = [make_y(c) for c in range(NC)]
        x_rdmas = [make_x(c) for c in range(NC)]
        z1_rdmas = [make_z(c, row0, c) for c in range(NC)]
        z2_rdmas = [make_z(c, row0x, NC + c) for c in range(NC)]

        def dy_copy(i):
            cols = pl.ds(i * DY_CHUNK, DY_CHUNK)
            return pltpu.make_async_copy(
                dy_ref.at[:, cols], dy_vmem.at[i % 2], dy_sems.at[i % 2])

        copies = [dy_copy(i) for i in range(N_DY)]
        copies[0].start()
        for i in range(N_DY):
            if i + 1 < N_DY:
                copies[i + 1].start()
            copies[i].wait()
            cols = pl.ds(i * DY_CHUNK, DY_CHUNK)
            d = dy_vmem[i % 2, :, :]
            send_buf[:, cols] = lax.dot_general(
                a_pr_ref[:, :], d, (((1,), (0,)), ((), ())),
                preferred_element_type=jnp.float32)
            out_ref[pl.ds(row0, M_BLK), cols] = lax.dot_general(
                a_me_ref[:, :], d, (((1,), (0,)), ((), ())),
                preferred_element_type=jnp.float32)
            if DO_Y and (i + 1) % SUB == 0:
                y_rdmas[(i + 1) // SUB - 1].start()

        LAG = 2
        for c in range(NC):
            cols = pl.ds(c * CW, CW)
            if DO_Y:
                y_rdmas[c].wait_recv()
                out_ref[pl.ds(row0, M_BLK), cols] = (
                    out_ref[pl.ds(row0, M_BLK), cols] + recv_buf[:, cols])
            if DO_Z:
                z1_rdmas[c].start()
            if DO_X:
                x_rdmas[c].start()
                if c >= LAG:
                    x_rdmas[c - LAG].wait_recv()
                    if DO_Z:
                        z2_rdmas[c - LAG].start()

        if DO_X:
            for c in range(NC - LAG, NC):
                x_rdmas[c].wait_recv()
                if DO_Z:
                    z2_rdmas[c].start()

        if DO_Z:
            for c in range(NC):
                z1_rdmas[c].wait_recv()
                if DO_X:
                    z2_rdmas[c].wait_recv()
        for c in range(NC):
            if DO_Y:
                y_rdmas[c].wait_send()
            if DO_X:
                x_rdmas[c].wait_send()
            if DO_Z:
                z1_rdmas[c].wait_send()
                if DO_X:
                    z2_rdmas[c].wait_send()

    return pl.pallas_call(
        body,
        out_shape=jax.ShapeDtypeStruct((M_SHARD, F), jnp.float32),
        in_specs=[
            pl.BlockSpec(memory_space=pltpu.VMEM),
            pl.BlockSpec(memory_space=pltpu.VMEM),
            pl.BlockSpec(memory_space=pl.ANY),
        ],
        out_specs=pl.BlockSpec(memory_space=pltpu.VMEM),
        scratch_shapes=[
            pltpu.VMEM((2, K, DY_CHUNK), jnp.float32),
            pltpu.VMEM((M_BLK, F), jnp.float32),
            pltpu.VMEM((M_BLK, F), jnp.float32),
            pltpu.SemaphoreType.DMA((2,)),
            pltpu.SemaphoreType.DMA((NC,)),
            pltpu.SemaphoreType.DMA((NC,)),
            pltpu.SemaphoreType.DMA((NC,)),
            pltpu.SemaphoreType.DMA((NC,)),
            pltpu.SemaphoreType.DMA((2 * NC,)),
            pltpu.SemaphoreType.DMA((2 * NC,)),
        ],
        compiler_params=pltpu.CompilerParams(
            collective_id=0, vmem_limit_bytes=64 * 1024 * 1024),
    )(a_me, a_pr, dy)


# device time: 144209 ns/iter; 1.8157x vs baseline; 1.0028x over previous
import os

import jax
import jax.numpy as jnp
from jax import lax
from jax.experimental import pallas as pl
from jax.experimental.pallas import tpu as pltpu

K = 2048
M_SHARD = 1024
M_BLK = 256
F = 8192
DY_CHUNK = 512
N_DY = F // DY_CHUNK
CW = 512
NC = F // CW
SUB = CW // DY_CHUNK

DO_Y = "y" not in os.environ.get("KSKIP", "")
DO_X = "x" not in os.environ.get("KSKIP", "")
DO_Z = "z" not in os.environ.get("KSKIP", "")


def kernel(x, dy):
    my_x = lax.axis_index("x")
    my_y = lax.axis_index("y")
    my_z = lax.axis_index("z")
    q = 2 * my_z + my_x

    col_me = my_y * M_SHARD + q * M_BLK
    col_pr = (1 - my_y) * M_SHARD + q * M_BLK
    a_me = lax.dynamic_slice(x, (0, col_me), (K, M_BLK)).T
    a_pr = lax.dynamic_slice(x, (0, col_pr), (K, M_BLK)).T

    def body(a_me_ref, a_pr_ref, dy_ref, out_ref,
             dy_vmem, send_buf, recv_buf,
             dy_sems, y_send_sems, y_recv_sems, x_send_sems, x_recv_sems,
             z_send_sems, z_recv_sems):
        my_x = lax.axis_index("x")
        my_y = lax.axis_index("y")
        my_z = lax.axis_index("z")
        q = 2 * my_z + my_x
        row0 = q * M_BLK
        row0x = (2 * my_z + (1 - my_x)) * M_BLK

        y_dev = (my_x, 1 - my_y, my_z)
        x_dev = (1 - my_x, my_y, my_z)
        z_dev = (my_x, my_y, 1 - my_z)

        barrier = pltpu.get_barrier_semaphore()
        for dev in (y_dev, x_dev, z_dev):
            pl.semaphore_signal(barrier, inc=1, device_id=dev,
                                device_id_type=pl.DeviceIdType.MESH)
        pl.semaphore_wait(barrier, 3)

        def make_y(c):
            cols = pl.ds(c * CW, CW)
            return pltpu.make_async_remote_copy(
                src_ref=send_buf.at[:, cols], dst_ref=recv_buf.at[:, cols],
                send_sem=y_send_sems.at[c], recv_sem=y_recv_sems.at[c],
                device_id=y_dev, device_id_type=pl.DeviceIdType.MESH)

        def make_x(c):
            cols = pl.ds(c * CW, CW)
            return pltpu.make_async_remote_copy(
                src_ref=out_ref.at[pl.ds(row0, M_BLK), cols],
                dst_ref=out_ref.at[pl.ds(row0, M_BLK), cols],
                send_sem=x_send_sems.at[c], recv_sem=x_recv_sems.at[c],
                device_id=x_dev, device_id_type=pl.DeviceIdType.MESH)

        def make_z(c, rows, sem_idx):
            cols = pl.ds(c * CW, CW)
            return pltpu.make_async_remote_copy(
                src_ref=out_ref.at[pl.ds(rows, M_BLK), cols],
                dst_ref=out_ref.at[pl.ds(rows, M_BLK), cols],
                send_sem=z_send_sems.at[sem_idx], recv_sem=z_recv_sems.at[sem_idx],
                device_id=z_dev, device_id_type=pl.DeviceIdType.MESH)

        y_rdmas = [make_y(c) for c in range(NC)]
        x_rdmas = [make_x(c) for c in range(NC)]
        z1_rdmas = [make_z(c, row0, c) for c in range(NC)]
        z2_rdmas = [make_z(c, row0x, NC + c) for c in range(NC)]

        def dy_copy(i):
            cols = pl.ds(i * DY_CHUNK, DY_CHUNK)
            return pltpu.make_async_copy(
                dy_ref.at[:, cols], dy_vmem.at[i % 2], dy_sems.at[i % 2])

        copies = [dy_copy(i) for i in range(N_DY)]
        copies[0].start()
        for i in range(N_DY):
            copies[i].wait()
            cols = pl.ds(i * DY_CHUNK, DY_CHUNK)
            d = dy_vmem[i % 2, :, :]
            send_buf[:, cols] = lax.dot_general(
                a_pr_ref[:, :], d, (((1,), (0,)), ((), ())),
                preferred_element_type=jnp.float32)
            if DO_Y and (i + 1) % SUB == 0:
                y_rdmas[(i + 1) // SUB - 1].start()
            if i + 1 < N_DY:
                copies[i + 1].start()
            out_ref[pl.ds(row0, M_BLK), cols] = lax.dot_general(
                a_me_ref[:, :], d, (((1,), (0,)), ((), ())),
                preferred_element_type=jnp.float32)

        LAG = 2
        for c in range(NC):
            cols = pl.ds(c * CW, CW)
            if DO_Y:
                y_rdmas[c].wait_recv()
                out_ref[pl.ds(row0, M_BLK), cols] = (
                    out_ref[pl.ds(row0, M_BLK), cols] + recv_buf[:, cols])
            if DO_Z:
                z1_rdmas[c].start()
            if DO_X:
                x_rdmas[c].start()
                if c >= LAG:
                    x_rdmas[c - LAG].wait_recv()
                    if DO_Z:
                        z2_rdmas[c - LAG].start()

        if DO_X:
            for c in range(NC - LAG, NC):
                x_rdmas[c].wait_recv()
                if DO_Z:
                    z2_rdmas[c].start()

        if DO_Z:
            for c in range(NC):
                z1_rdmas[c].wait_recv()
                if DO_X:
                    z2_rdmas[c].wait_recv()
        for c in range(NC):
            if DO_Y:
                y_rdmas[c].wait_send()
            if DO_X:
                x_rdmas[c].wait_send()
            if DO_Z:
                z1_rdmas[c].wait_send()
                if DO_X:
                    z2_rdmas[c].wait_send()

    return pl.pallas_call(
        body,
        out_shape=jax.ShapeDtypeStruct((M_SHARD, F), jnp.float32),
        in_specs=[
            pl.BlockSpec(memory_space=pltpu.VMEM),
            pl.BlockSpec(memory_space=pltpu.VMEM),
            pl.BlockSpec(memory_space=pl.ANY),
        ],
        out_specs=pl.BlockSpec(memory_space=pltpu.VMEM),
        scratch_shapes=[
            pltpu.VMEM((2, K, DY_CHUNK), jnp.float32),
            pltpu.VMEM((M_BLK, F), jnp.float32),
            pltpu.VMEM((M_BLK, F), jnp.float32),
            pltpu.SemaphoreType.DMA((2,)),
            pltpu.SemaphoreType.DMA((NC,)),
            pltpu.SemaphoreType.DMA((NC,)),
            pltpu.SemaphoreType.DMA((NC,)),
            pltpu.SemaphoreType.DMA((NC,)),
            pltpu.SemaphoreType.DMA((2 * NC,)),
            pltpu.SemaphoreType.DMA((2 * NC,)),
        ],
        compiler_params=pltpu.CompilerParams(
            collective_id=0, vmem_limit_bytes=64 * 1024 * 1024),
    )(a_me, a_pr, dy)


# device time: 140697 ns/iter; 1.8610x vs baseline; 1.0250x over previous
import os

import jax
import jax.numpy as jnp
from jax import lax
from jax.experimental import pallas as pl
from jax.experimental.pallas import tpu as pltpu

K = 2048
M_SHARD = 1024
M_BLK = 256
F = 8192
DY_CHUNK = 512
N_DY = F // DY_CHUNK
CW = 512
NC = F // CW
SUB = CW // DY_CHUNK

DO_Y = "y" not in os.environ.get("KSKIP", "")
DO_X = "x" not in os.environ.get("KSKIP", "")
DO_Z = "z" not in os.environ.get("KSKIP", "")
DO_GEMM = not os.environ.get("KNOGEMM")
DO_ADD = not os.environ.get("KNOADD")


def kernel(x, dy):
    my_x = lax.axis_index("x")
    my_y = lax.axis_index("y")
    my_z = lax.axis_index("z")
    q = 2 * my_z + my_x

    col_me = my_y * M_SHARD + q * M_BLK
    col_pr = (1 - my_y) * M_SHARD + q * M_BLK
    a_me = lax.dynamic_slice(x, (0, col_me), (K, M_BLK)).T
    a_pr = lax.dynamic_slice(x, (0, col_pr), (K, M_BLK)).T

    def body(a_me_ref, a_pr_ref, dy_ref, out_ref,
             dy_vmem, send_buf, recv_buf,
             dy_sems, y_send_sems, y_recv_sems, x_send_sems, x_recv_sems,
             z_send_sems, z_recv_sems):
        my_x = lax.axis_index("x")
        my_y = lax.axis_index("y")
        my_z = lax.axis_index("z")
        q = 2 * my_z + my_x
        row0 = q * M_BLK
        row0x = (2 * my_z + (1 - my_x)) * M_BLK

        y_dev = (my_x, 1 - my_y, my_z)
        x_dev = (1 - my_x, my_y, my_z)
        z_dev = (my_x, my_y, 1 - my_z)

        barrier = pltpu.get_barrier_semaphore()
        for dev in (y_dev, x_dev, z_dev):
            pl.semaphore_signal(barrier, inc=1, device_id=dev,
                                device_id_type=pl.DeviceIdType.MESH)
        pl.semaphore_wait(barrier, 3)

        def make_y(c):
            cols = pl.ds(c * CW, CW)
            return pltpu.make_async_remote_copy(
                src_ref=send_buf.at[:, cols], dst_ref=recv_buf.at[:, cols],
                send_sem=y_send_sems.at[c], recv_sem=y_recv_sems.at[c],
                device_id=y_dev, device_id_type=pl.DeviceIdType.MESH)

        def make_x(c):
            cols = pl.ds(c * CW, CW)
            return pltpu.make_async_remote_copy(
                src_ref=out_ref.at[pl.ds(row0, M_BLK), cols],
                dst_ref=out_ref.at[pl.ds(row0, M_BLK), cols],
                send_sem=x_send_sems.at[c], recv_sem=x_recv_sems.at[c],
                device_id=x_dev, device_id_type=pl.DeviceIdType.MESH)

        def make_z(c, rows, sem_idx):
            cols = pl.ds(c * CW, CW)
            return pltpu.make_async_remote_copy(
                src_ref=out_ref.at[pl.ds(rows, M_BLK), cols],
                dst_ref=out_ref.at[pl.ds(rows, M_BLK), cols],
                send_sem=z_send_sems.at[sem_idx], recv_sem=z_recv_sems.at[sem_idx],
                device_id=z_dev, device_id_type=pl.DeviceIdType.MESH)

        y_rdmas = [make_y(c) for c in range(NC)]
        x_rdmas = [make_x(c) for c in range(NC)]
        z1_rdmas = [make_z(c, row0, c) for c in range(NC)]
        z2_rdmas = [make_z(c, row0x, NC + c) for c in range(NC)]

        def dy_copy(i):
            cols = pl.ds(i * DY_CHUNK, DY_CHUNK)
            return pltpu.make_async_copy(
                dy_ref.at[:, cols], dy_vmem.at[i % 2], dy_sems.at[i % 2])

        if DO_GEMM:
            copies = [dy_copy(i) for i in range(N_DY)]
            copies[0].start()
            for i in range(N_DY):
                copies[i].wait()
                cols = pl.ds(i * DY_CHUNK, DY_CHUNK)
                d = dy_vmem[i % 2, :, :]
                send_buf[:, cols] = lax.dot_general(
                    a_pr_ref[:, :], d, (((1,), (0,)), ((), ())),
                    preferred_element_type=jnp.float32)
                if DO_Y and (i + 1) % SUB == 0:
                    y_rdmas[(i + 1) // SUB - 1].start()
                if i + 1 < N_DY:
                    copies[i + 1].start()
                out_ref[pl.ds(row0, M_BLK), cols] = lax.dot_general(
                    a_me_ref[:, :], d, (((1,), (0,)), ((), ())),
                    preferred_element_type=jnp.float32)
        elif DO_Y:
            for c in range(NC):
                y_rdmas[c].start()

        LAG = 2
        for c in range(NC):
            cols = pl.ds(c * CW, CW)
            if DO_Y:
                y_rdmas[c].wait_recv()
                if DO_ADD:
                    out_ref[pl.ds(row0, M_BLK), cols] = (
                        out_ref[pl.ds(row0, M_BLK), cols] + recv_buf[:, cols])
            if DO_Z:
                z1_rdmas[c].start()
            if DO_X:
                x_rdmas[c].start()
                if c >= LAG:
                    x_rdmas[c - LAG].wait_recv()
                    if DO_Z:
                        z2_rdmas[c - LAG].start()

        if DO_X:
            for c in range(NC - LAG, NC):
                x_rdmas[c].wait_recv()
                if DO_Z:
                    z2_rdmas[c].start()

        if DO_Z:
            for c in range(NC):
                z1_rdmas[c].wait_recv()
                if DO_X:
                    z2_rdmas[c].wait_recv()
        for c in range(NC):
            if DO_Y:
                y_rdmas[c].wait_send()
            if DO_X:
                x_rdmas[c].wait_send()
            if DO_Z:
                z1_rdmas[c].wait_send()
                if DO_X:
                    z2_rdmas[c].wait_send()

    return pl.pallas_call(
        body,
        out_shape=jax.ShapeDtypeStruct((M_SHARD, F), jnp.float32),
        in_specs=[
            pl.BlockSpec(memory_space=pltpu.VMEM),
            pl.BlockSpec(memory_space=pltpu.VMEM),
            pl.BlockSpec(memory_space=pl.ANY),
        ],
        out_specs=pl.BlockSpec(memory_space=pltpu.VMEM),
        scratch_shapes=[
            pltpu.VMEM((2, K, DY_CHUNK), jnp.float32),
            pltpu.VMEM((M_BLK, F), jnp.float32),
            pltpu.VMEM((M_BLK, F), jnp.float32),
            pltpu.SemaphoreType.DMA((2,)),
            pltpu.SemaphoreType.DMA((NC,)),
            pltpu.SemaphoreType.DMA((NC,)),
            pltpu.SemaphoreType.DMA((NC,)),
            pltpu.SemaphoreType.DMA((NC,)),
            pltpu.SemaphoreType.DMA((2 * NC,)),
            pltpu.SemaphoreType.DMA((2 * NC,)),
        ],
        compiler_params=pltpu.CompilerParams(
            collective_id=0, vmem_limit_bytes=64 * 1024 * 1024),
    )(a_me, a_pr, dy)
